# baseline (speedup 1.0000x reference)
import functools

import jax
import jax.numpy as jnp
from jax import lax
from jax.experimental import pallas as pl
from jax.experimental.pallas import tpu as pltpu
from jax.experimental.pallas import tpu_sc as plsc

B, S, H = 4, 2048, 2048
HR = 1024
E = 16
N = B * S


_TBLK = 512
_NBLK = N // _TBLK


def _router_body(flat_ref, w1_ref, b1_ref, w2_ref, b2_ref, idx_ref):
    x = flat_ref[...]
    h = jnp.maximum(jnp.dot(x, w1_ref[...], preferred_element_type=jnp.float32)
                    + b1_ref[...], 0.0)
    s = jnp.dot(h, w2_ref[...], preferred_element_type=jnp.float32) + b2_ref[...]
    m = jnp.max(s, axis=-1, keepdims=True)
    iota = lax.broadcasted_iota(jnp.int32, s.shape, 1)
    idx = jnp.min(jnp.where(s == m, iota, E), axis=-1)
    idx_ref[...] = idx.reshape(1, 1, _TBLK)


def _router_idx(flat, w1, b1, w2, b2):
    n = flat.shape[0]
    nblk = n // _TBLK
    out = pl.pallas_call(
        _router_body,
        grid=(nblk,),
        in_specs=[
            pl.BlockSpec((_TBLK, H), lambda i: (i, 0)),
            pl.BlockSpec((H, HR), lambda i: (0, 0)),
            pl.BlockSpec((1, HR), lambda i: (0, 0)),
            pl.BlockSpec((HR, E), lambda i: (0, 0)),
            pl.BlockSpec((1, E), lambda i: (0, 0)),
        ],
        out_specs=pl.BlockSpec((1, 1, _TBLK), lambda i: (i, 0, 0)),
        out_shape=jax.ShapeDtypeStruct((nblk, 1, _TBLK), jnp.int32),
    )(flat, w1, b1.reshape(1, HR), w2, b2.reshape(1, E))
    return out.reshape(n)


_NW = 32
_TPW = N // _NW
_CH = 8
_NCH = _TPW // _CH
_UNROLL = 8


def _route_sc_body(tpw, nch, flat_hbm, idx_hbm, sb_hbm, out_hbm,
                   idx_v, sb_all, flat0, flat1, sin0, sin1, sout0, sout1):
    flat_b = (flat0, flat1)
    sin = (sin0, sin1)
    sout = (sout0, sout1)

    wid = lax.axis_index("s") * 2 + lax.axis_index("c")
    base = wid * tpw
    pltpu.sync_copy(idx_hbm.at[pl.ds(base, tpw)], idx_v)
    pltpu.sync_copy(sb_hbm, sb_all)
    iota = lax.iota(jnp.int32, 16)

    def issue_in(c, b):
        pltpu.async_copy(flat_hbm.at[pl.ds(base + c * _CH, _CH)],
                         flat_b[b], sin[b])

    issue_in(0, 0)
    issue_in(1, 1)

    def pair(p, _):
        for b in range(2):
            c = p * 2 + b
            pltpu.make_async_copy(flat_hbm.at[pl.ds(base, _CH)],
                                  flat_b[b], sin[b]).wait()
            for t in range(_CH):
                bcast = plsc.load_gather(
                    idx_v, [jnp.full((16,), c * _CH + t, jnp.int32)])
                rb = bcast * (2 * H) + iota

                @plsc.parallel_loop(0, H // 16, unroll=_UNROLL)
                def _(j, t=t, rb=rb):
                    o = j * 16
                    isc = rb + o
                    sc = plsc.load_gather(sb_all, [isc])
                    bi = plsc.load_gather(sb_all, [isc + H])
                    f = flat_b[b][t, pl.ds(o, 16)]
                    flat_b[b][t, pl.ds(o, 16)] = f * sc + bi
            tb = base + c * _CH
            pltpu.async_copy(flat_b[b], out_hbm.at[pl.ds(tb, _CH)], sout[b])
            pltpu.make_async_copy(flat_b[b], out_hbm.at[pl.ds(base, _CH)],
                                  sout[b]).wait()

            @pl.when(c + 2 < nch)
            def _():
                issue_in(c + 2, b)
        return 0

    lax.fori_loop(0, nch // 2, pair, 0)


def _route_sc(flat, idx, sb_cat):
    n = flat.shape[0]
    tpw = n // _NW
    nch = tpw // _CH
    mesh = plsc.VectorSubcoreMesh(core_axis_name="c", subcore_axis_name="s")
    f = pl.kernel(
        functools.partial(_route_sc_body, tpw, nch),
        mesh=mesh,
        compiler_params=pltpu.CompilerParams(needs_layout_passes=False),
        out_type=jax.ShapeDtypeStruct((n, H), jnp.float32),
        scratch_types=[
            pltpu.VMEM((tpw,), jnp.int32),
            pltpu.VMEM((E * 2 * H,), jnp.float32),
            pltpu.VMEM((_CH, H), jnp.float32),
            pltpu.VMEM((_CH, H), jnp.float32),
            pltpu.SemaphoreType.DMA,
            pltpu.SemaphoreType.DMA,
            pltpu.SemaphoreType.DMA,
            pltpu.SemaphoreType.DMA,
        ],
    )
    return f(flat, idx, sb_cat)


def _pack_sb(scales, biases):
    return jnp.concatenate([scales, biases], axis=1).reshape(E * 2 * H)


_NSPLIT = 2


def kernel(hidden_states, W1, b1, W2, b2, Wg1, bg1, Wg2, bg2,
           expert_scales, expert_biases):
    flat = hidden_states.reshape(N, H)
    sb = _pack_sb(expert_scales, expert_biases)
    np = N // _NSPLIT
    parts = []
    for p in range(_NSPLIT):
        fp = lax.slice_in_dim(flat, p * np, (p + 1) * np, axis=0)
        idx = _router_idx(fp, W1, b1, W2, b2)
        parts.append(_route_sc(fp, idx, sb))
    routed = jnp.concatenate(parts, axis=0)
    return routed.reshape(B, S, H)


# speedup vs baseline: 1.7508x; 1.7508x over previous
import functools

import jax
import jax.numpy as jnp
from jax import lax
from jax.experimental import pallas as pl
from jax.experimental.pallas import tpu as pltpu
from jax.experimental.pallas import tpu_sc as plsc

B, S, H = 4, 2048, 2048
HR = 1024
E = 16
N = B * S


_TBLK = 512
_NBLK = N // _TBLK


def _router_body(flat_ref, w1_ref, b1_ref, w2_ref, b2_ref, idx_ref):
    x = flat_ref[...]
    h = jnp.maximum(jnp.dot(x, w1_ref[...], preferred_element_type=jnp.float32)
                    + b1_ref[...], 0.0)
    s = jnp.dot(h, w2_ref[...], preferred_element_type=jnp.float32) + b2_ref[...]
    m = jnp.max(s, axis=-1, keepdims=True)
    iota = lax.broadcasted_iota(jnp.int32, s.shape, 1)
    idx = jnp.min(jnp.where(s == m, iota, E), axis=-1)
    idx_ref[...] = idx.reshape(1, 1, _TBLK)


def _router_idx(flat, w1, b1, w2, b2):
    n = flat.shape[0]
    nblk = n // _TBLK
    out = pl.pallas_call(
        _router_body,
        grid=(nblk,),
        in_specs=[
            pl.BlockSpec((_TBLK, H), lambda i: (i, 0)),
            pl.BlockSpec((H, HR), lambda i: (0, 0)),
            pl.BlockSpec((1, HR), lambda i: (0, 0)),
            pl.BlockSpec((HR, E), lambda i: (0, 0)),
            pl.BlockSpec((1, E), lambda i: (0, 0)),
        ],
        out_specs=pl.BlockSpec((1, 1, _TBLK), lambda i: (i, 0, 0)),
        out_shape=jax.ShapeDtypeStruct((nblk, 1, _TBLK), jnp.int32),
    )(flat, w1, b1.reshape(1, HR), w2, b2.reshape(1, E))
    return out.reshape(n)


_NW = 32
_TPW = N // _NW
_CH = 8
_NCH = _TPW // _CH
_UNROLL = 8


def _route_sc_body(tpw, nch, flat_hbm, idx_hbm, sb_hbm, out_hbm,
                   idx_v, sb_all, flat0, flat1, out0, out1,
                   sin0, sin1, sout0, sout1):
    flat_b = (flat0, flat1)
    out_b = (out0, out1)
    sin = (sin0, sin1)
    sout = (sout0, sout1)

    wid = lax.axis_index("s") * 2 + lax.axis_index("c")
    base = wid * tpw
    pltpu.sync_copy(idx_hbm.at[pl.ds(base, tpw)], idx_v)
    pltpu.sync_copy(sb_hbm, sb_all)
    iota = lax.iota(jnp.int32, 16)
    himask = jnp.full((16,), -65536, jnp.int32)

    def issue_in(c, b):
        pltpu.async_copy(flat_hbm.at[pl.ds(base + c * _CH, _CH)],
                         flat_b[b], sin[b])

    issue_in(0, 0)
    issue_in(1, 1)

    def pair(p, _):
        for b in range(2):
            c = p * 2 + b
            pltpu.make_async_copy(flat_hbm.at[pl.ds(base, _CH)],
                                  flat_b[b], sin[b]).wait()

            @pl.when(c >= 2)
            def _():
                pltpu.make_async_copy(out_b[b],
                                      out_hbm.at[pl.ds(base, _CH)],
                                      sout[b]).wait()

            for t in range(_CH):
                bcast = plsc.load_gather(
                    idx_v, [jnp.full((16,), c * _CH + t, jnp.int32)])
                rb = bcast * H + iota

                @plsc.parallel_loop(0, H // 16, unroll=_UNROLL)
                def _(j, t=t, rb=rb):
                    o = j * 16
                    w = plsc.load_gather(sb_all, [rb + o])
                    sc = plsc.bitcast(w & himask, jnp.float32)
                    bi = plsc.bitcast(lax.shift_left(w, 16), jnp.float32)
                    f = flat_b[b][t, pl.ds(o, 16)]
                    out_b[b][t, pl.ds(o, 16)] = f * sc + bi

            tb = base + c * _CH
            pltpu.async_copy(out_b[b], out_hbm.at[pl.ds(tb, _CH)], sout[b])

            @pl.when(c + 2 < nch)
            def _():
                issue_in(c + 2, b)
        return 0

    lax.fori_loop(0, nch // 2, pair, 0)
    for b in range(2):
        pltpu.make_async_copy(out_b[b], out_hbm.at[pl.ds(base, _CH)],
                              sout[b]).wait()


def _route_sc(flat, idx, sb_packed):
    n = flat.shape[0]
    tpw = n // _NW
    nch = tpw // _CH
    mesh = plsc.VectorSubcoreMesh(core_axis_name="c", subcore_axis_name="s")
    f = pl.kernel(
        functools.partial(_route_sc_body, tpw, nch),
        mesh=mesh,
        compiler_params=pltpu.CompilerParams(needs_layout_passes=False),
        out_type=jax.ShapeDtypeStruct((n, H), jnp.float32),
        scratch_types=[
            pltpu.VMEM((tpw,), jnp.int32),
            pltpu.VMEM((E * H,), jnp.int32),
            pltpu.VMEM((_CH, H), jnp.float32),
            pltpu.VMEM((_CH, H), jnp.float32),
            pltpu.VMEM((_CH, H), jnp.float32),
            pltpu.VMEM((_CH, H), jnp.float32),
            pltpu.SemaphoreType.DMA,
            pltpu.SemaphoreType.DMA,
            pltpu.SemaphoreType.DMA,
            pltpu.SemaphoreType.DMA,
        ],
    )
    return f(flat, idx, sb_packed)


def _pack_sb(scales, biases):
    s16 = lax.bitcast_convert_type(scales.astype(jnp.bfloat16), jnp.uint16)
    b16 = lax.bitcast_convert_type(biases.astype(jnp.bfloat16), jnp.uint16)
    w = (s16.astype(jnp.uint32) << 16) | b16.astype(jnp.uint32)
    return lax.bitcast_convert_type(w, jnp.int32).reshape(E * H)


def kernel(hidden_states, W1, b1, W2, b2, Wg1, bg1, Wg2, bg2,
           expert_scales, expert_biases):
    flat = hidden_states.reshape(N, H)
    idx = _router_idx(flat, W1, b1, W2, b2)
    routed = _route_sc(flat, idx, _pack_sb(expert_scales, expert_biases))
    return routed.reshape(B, S, H)


# speedup vs baseline: 1.7588x; 1.0046x over previous
import functools

import jax
import jax.numpy as jnp
from jax import lax
from jax.experimental import pallas as pl
from jax.experimental.pallas import tpu as pltpu
from jax.experimental.pallas import tpu_sc as plsc

B, S, H = 4, 2048, 2048
HR = 1024
E = 16
N = B * S


_TBLK = 512
_NBLK = N // _TBLK


def _router_body(flat_ref, w1_ref, b1_ref, w2_ref, b2_ref, idx_ref):
    x = flat_ref[...].astype(jnp.bfloat16)
    h = jnp.maximum(
        jnp.dot(x, w1_ref[...].astype(jnp.bfloat16),
                preferred_element_type=jnp.float32) + b1_ref[...], 0.0)
    s = jnp.dot(h.astype(jnp.bfloat16), w2_ref[...].astype(jnp.bfloat16),
                preferred_element_type=jnp.float32) + b2_ref[...]
    m = jnp.max(s, axis=-1, keepdims=True)
    iota = lax.broadcasted_iota(jnp.int32, s.shape, 1)
    idx = jnp.min(jnp.where(s == m, iota, E), axis=-1)
    idx_ref[...] = idx.reshape(1, 1, _TBLK)


def _router_idx(flat, w1, b1, w2, b2):
    n = flat.shape[0]
    nblk = n // _TBLK
    out = pl.pallas_call(
        _router_body,
        grid=(nblk,),
        in_specs=[
            pl.BlockSpec((_TBLK, H), lambda i: (i, 0)),
            pl.BlockSpec((H, HR), lambda i: (0, 0)),
            pl.BlockSpec((1, HR), lambda i: (0, 0)),
            pl.BlockSpec((HR, E), lambda i: (0, 0)),
            pl.BlockSpec((1, E), lambda i: (0, 0)),
        ],
        out_specs=pl.BlockSpec((1, 1, _TBLK), lambda i: (i, 0, 0)),
        out_shape=jax.ShapeDtypeStruct((nblk, 1, _TBLK), jnp.int32),
    )(flat, w1, b1.reshape(1, HR), w2, b2.reshape(1, E))
    return out.reshape(n)


_NW = 32
_TPW = N // _NW
_CH = 8
_NCH = _TPW // _CH
_UNROLL = 8


def _route_sc_body(tpw, nch, flat_hbm, idx_hbm, sb_hbm, out_hbm,
                   idx_v, sb_all, flat0, flat1, out0, out1,
                   sin0, sin1, sout0, sout1):
    flat_b = (flat0, flat1)
    out_b = (out0, out1)
    sin = (sin0, sin1)
    sout = (sout0, sout1)

    wid = lax.axis_index("s") * 2 + lax.axis_index("c")
    base = wid * tpw
    pltpu.sync_copy(idx_hbm.at[pl.ds(base, tpw)], idx_v)
    pltpu.sync_copy(sb_hbm, sb_all)
    iota = lax.iota(jnp.int32, 16)
    himask = jnp.full((16,), -65536, jnp.int32)

    def issue_in(c, b):
        pltpu.async_copy(flat_hbm.at[pl.ds(base + c * _CH, _CH)],
                         flat_b[b], sin[b])

    issue_in(0, 0)
    issue_in(1, 1)

    def pair(p, _):
        for b in range(2):
            c = p * 2 + b
            pltpu.make_async_copy(flat_hbm.at[pl.ds(base, _CH)],
                                  flat_b[b], sin[b]).wait()

            @pl.when(c >= 2)
            def _():
                pltpu.make_async_copy(out_b[b],
                                      out_hbm.at[pl.ds(base, _CH)],
                                      sout[b]).wait()

            for t in range(_CH):
                bcast = plsc.load_gather(
                    idx_v, [jnp.full((16,), c * _CH + t, jnp.int32)])
                rb = bcast * H + iota

                @plsc.parallel_loop(0, H // 16, unroll=_UNROLL)
                def _(j, t=t, rb=rb):
                    o = j * 16
                    w = plsc.load_gather(sb_all, [rb + o])
                    sc = plsc.bitcast(w & himask, jnp.float32)
                    bi = plsc.bitcast(lax.shift_left(w, 16), jnp.float32)
                    f = flat_b[b][t, pl.ds(o, 16)]
                    out_b[b][t, pl.ds(o, 16)] = f * sc + bi

            tb = base + c * _CH
            pltpu.async_copy(out_b[b], out_hbm.at[pl.ds(tb, _CH)], sout[b])

            @pl.when(c + 2 < nch)
            def _():
                issue_in(c + 2, b)
        return 0

    lax.fori_loop(0, nch // 2, pair, 0)
    for b in range(2):
        pltpu.make_async_copy(out_b[b], out_hbm.at[pl.ds(base, _CH)],
                              sout[b]).wait()


def _route_sc(flat, idx, sb_packed):
    n = flat.shape[0]
    tpw = n // _NW
    nch = tpw // _CH
    mesh = plsc.VectorSubcoreMesh(core_axis_name="c", subcore_axis_name="s")
    f = pl.kernel(
        functools.partial(_route_sc_body, tpw, nch),
        mesh=mesh,
        compiler_params=pltpu.CompilerParams(needs_layout_passes=False),
        out_type=jax.ShapeDtypeStruct((n, H), jnp.float32),
        scratch_types=[
            pltpu.VMEM((tpw,), jnp.int32),
            pltpu.VMEM((E * H,), jnp.int32),
            pltpu.VMEM((_CH, H), jnp.float32),
            pltpu.VMEM((_CH, H), jnp.float32),
            pltpu.VMEM((_CH, H), jnp.float32),
            pltpu.VMEM((_CH, H), jnp.float32),
            pltpu.SemaphoreType.DMA,
            pltpu.SemaphoreType.DMA,
            pltpu.SemaphoreType.DMA,
            pltpu.SemaphoreType.DMA,
        ],
    )
    return f(flat, idx, sb_packed)


def _pack_sb(scales, biases):
    s16 = lax.bitcast_convert_type(scales.astype(jnp.bfloat16), jnp.uint16)
    b16 = lax.bitcast_convert_type(biases.astype(jnp.bfloat16), jnp.uint16)
    w = (s16.astype(jnp.uint32) << 16) | b16.astype(jnp.uint32)
    return lax.bitcast_convert_type(w, jnp.int32).reshape(E * H)


def kernel(hidden_states, W1, b1, W2, b2, Wg1, bg1, Wg2, bg2,
           expert_scales, expert_biases):
    flat = hidden_states.reshape(N, H)
    idx = _router_idx(flat, W1, b1, W2, b2)
    routed = _route_sc(flat, idx, _pack_sb(expert_scales, expert_biases))
    return routed.reshape(B, S, H)
